# baseline (speedup 1.0000x reference)
import functools

import jax
import jax.numpy as jnp
from jax import lax
from jax.experimental import pallas as pl
from jax.experimental.pallas import tpu as pltpu
from jax.experimental.pallas import tpu_sc as plsc

B = 16384
F = 26
V = 100000
D = 16
H1 = 16
H2 = 32
EPS = 1e-5

total = F * V
BF = B * F
NW = 32
ROWS_W = BF // NW
G = 128
NG = ROWS_W // G
NCHUNK = 8
GPC = NG // NCHUNK
CROWS = GPC * G


def _sc_gather(idx2d, emb_w, lin_w):
  mesh = plsc.VectorSubcoreMesh(core_axis_name="c", subcore_axis_name="s")

  @functools.partial(
      pl.kernel,
      out_type=[
          jax.ShapeDtypeStruct((BF, D), jnp.float32),
          jax.ShapeDtypeStruct((BF,), jnp.float32),
      ],
      mesh=mesh,
      compiler_params=pltpu.CompilerParams(use_tc_tiling_on_sc=False),
      scratch_types=[
          pltpu.VMEM((NG, G), jnp.int32),
          pltpu.VMEM((2 * CROWS, D), jnp.float32),
          pltpu.VMEM((2 * CROWS,), jnp.float32),
          pltpu.SemaphoreType.DMA,
      ],
  )
  def body(idx_hbm, emb_hbm, lin_hbm, e_out, linv_out, idx_v, rows_v, lrows_v,
           sem):
    wid = lax.axis_index("s") * 2 + lax.axis_index("c")
    wbase = wid * ROWS_W
    pltpu.sync_copy(idx_hbm.at[pl.ds(wid * NG, NG), :], idx_v)

    def fire(c, buf):
      copies = []
      for j in range(GPC):
        irow = idx_v.at[c * GPC + j]
        dst = pl.ds(buf * CROWS + j * G, G)
        copies.append(
            pltpu.async_copy(emb_hbm.at[irow], rows_v.at[dst, :], sem))
        copies.append(
            pltpu.async_copy(lin_hbm.at[irow], lrows_v.at[dst], sem))
      return copies

    def drain(copies):
      for cp in copies:
        cp.wait()

    def writeout(c, buf):
      src = pl.ds(buf * CROWS, CROWS)
      out_rows = pl.ds(wbase + c * CROWS, CROWS)
      pltpu.sync_copy(rows_v.at[src, :], e_out.at[out_rows, :])
      pltpu.sync_copy(lrows_v.at[src], linv_out.at[out_rows])

    inflight = fire(0, 0)
    for c in range(NCHUNK):
      drain(inflight)
      if c + 1 < NCHUNK:
        nxt = fire(c + 1, (c + 1) % 2)
      writeout(c, c % 2)
      if c + 1 < NCHUNK:
        inflight = nxt

  return body(idx2d, emb_w, lin_w)


def _tc_body(e_ref, linv_ref, s_ref, w1_ref, b1_ref, w2_ref, b2_ref, w3_ref,
             c0_ref, out_ref):
  e = e_ref[...]
  s = jnp.dot(e, s_ref[...], preferred_element_type=jnp.float32)
  sumsq = jnp.sum(e * e, axis=1, keepdims=True)
  second = 0.5 * (jnp.sum(s * s, axis=1, keepdims=True) - sumsq)
  h = jnp.dot(e, w1_ref[...], preferred_element_type=jnp.float32) + b1_ref[...]
  h = jnp.maximum(h, 0.0)
  h = jnp.dot(h, w2_ref[...], preferred_element_type=jnp.float32) + b2_ref[...]
  h = jnp.maximum(h, 0.0)
  deep = jnp.dot(h, w3_ref[...], preferred_element_type=jnp.float32)
  first = jnp.sum(linv_ref[...], axis=1, keepdims=True)
  out_ref[...] = first + second + deep + c0_ref[0, 0]


def _tc_fused(e, linv, sel, w1f, b1f, w2f, b2f, w3, c0):
  bm = 2048
  grid = (B // bm,)
  return pl.pallas_call(
      _tc_body,
      grid=grid,
      in_specs=[
          pl.BlockSpec((bm, F * D), lambda i: (i, 0)),
          pl.BlockSpec((bm, F), lambda i: (i, 0)),
          pl.BlockSpec((F * D, D), lambda i: (0, 0)),
          pl.BlockSpec((F * D, H1), lambda i: (0, 0)),
          pl.BlockSpec((1, H1), lambda i: (0, 0)),
          pl.BlockSpec((H1, H2), lambda i: (0, 0)),
          pl.BlockSpec((1, H2), lambda i: (0, 0)),
          pl.BlockSpec((H2, 1), lambda i: (0, 0)),
          pl.BlockSpec((1, 1), lambda i: (0, 0)),
      ],
      out_specs=pl.BlockSpec((bm, 1), lambda i: (i, 0)),
      out_shape=jax.ShapeDtypeStruct((B, 1), jnp.float32),
  )(e, linv, sel, w1f, b1f, w2f, b2f, w3, c0)


def kernel(x, emb_w, lin_w, lin_b, W1, b1, g1, be1, rm1, rv1, W2, b2, g2, be2,
           rm2, rv2, W3, b3):
  offsets = (jnp.arange(F, dtype=x.dtype) * V)[None, :]
  idx2d = (x + offsets).reshape(BF // G, G)

  e_flat, linv_flat = _sc_gather(idx2d, emb_w, lin_w.reshape(total))
  e = e_flat.reshape(B, F * D)
  linv = linv_flat.reshape(B, F)

  inv1 = g1 / jnp.sqrt(rv1 + EPS)
  w1f = W1 * inv1[None, :]
  b1f = ((b1 - rm1) * inv1 + be1)[None, :]
  inv2 = g2 / jnp.sqrt(rv2 + EPS)
  w2f = W2 * inv2[None, :]
  b2f = ((b2 - rm2) * inv2 + be2)[None, :]
  sel = jnp.tile(jnp.eye(D, dtype=jnp.float32), (F, 1))
  c0 = (lin_b + b3).reshape(1, 1)

  out = _tc_fused(e, linv, sel, w1f, b1f, w2f, b2f, W3, c0)
  return out.reshape(B)


# speedup vs baseline: 1.0020x; 1.0020x over previous
import functools

import jax
import jax.numpy as jnp
from jax import lax
from jax.experimental import pallas as pl
from jax.experimental.pallas import tpu as pltpu
from jax.experimental.pallas import tpu_sc as plsc

B = 16384
F = 26
V = 100000
D = 16
H1 = 16
H2 = 32
EPS = 1e-5

total = F * V
BF = B * F
NW = 32
ROWS_W = BF // NW
G = 128
NG = ROWS_W // G
NCHUNK = 8
GPC = NG // NCHUNK
CROWS = GPC * G


def _sc_gather(idx2d, emb_w, lin_w):
  mesh = plsc.VectorSubcoreMesh(core_axis_name="c", subcore_axis_name="s")

  @functools.partial(
      pl.kernel,
      out_type=[
          jax.ShapeDtypeStruct((BF, D), jnp.float32),
          jax.ShapeDtypeStruct((BF,), jnp.float32),
      ],
      mesh=mesh,
      compiler_params=pltpu.CompilerParams(use_tc_tiling_on_sc=False),
      scratch_types=[
          pltpu.VMEM((NG, G), jnp.int32),
          pltpu.VMEM((2 * CROWS, D), jnp.float32),
          pltpu.VMEM((2 * CROWS,), jnp.float32),
          pltpu.SemaphoreType.DMA,
      ],
  )
  def body(idx_hbm, emb_hbm, lin_hbm, e_out, linv_out, idx_v, rows_v, lrows_v,
           sem):
    wid = lax.axis_index("s") * 2 + lax.axis_index("c")
    wbase = wid * ROWS_W
    pltpu.sync_copy(idx_hbm.at[pl.ds(wid * NG, NG), :], idx_v)

    def fire(c, buf):
      copies = []
      for j in range(GPC):
        irow = idx_v.at[c * GPC + j]
        dst = pl.ds(buf * CROWS + j * G, G)
        copies.append(
            pltpu.async_copy(emb_hbm.at[irow], rows_v.at[dst, :], sem))
        copies.append(
            pltpu.async_copy(lin_hbm.at[irow], lrows_v.at[dst], sem))
      return copies

    def drain(copies):
      for cp in copies:
        cp.wait()

    def writeout(c, buf):
      src = pl.ds(buf * CROWS, CROWS)
      out_rows = pl.ds(wbase + c * CROWS, CROWS)
      pltpu.sync_copy(rows_v.at[src, :], e_out.at[out_rows, :])
      pltpu.sync_copy(lrows_v.at[src], linv_out.at[out_rows])

    inflight = fire(0, 0)
    for c in range(NCHUNK):
      drain(inflight)
      if c + 1 < NCHUNK:
        nxt = fire(c + 1, (c + 1) % 2)
      writeout(c, c % 2)
      if c + 1 < NCHUNK:
        inflight = nxt

  return body(idx2d, emb_w, lin_w)


def _tc_body(e_ref, linv_ref, s_ref, w1_ref, b1_ref, w2_ref, b2_ref, w3_ref,
             c0_ref, out_ref):
  e = e_ref[...]
  s = jnp.dot(e, s_ref[...], preferred_element_type=jnp.float32)
  sumsq = jnp.sum(e * e, axis=1, keepdims=True)
  second = 0.5 * (jnp.sum(s * s, axis=1, keepdims=True) - sumsq)
  h = jnp.dot(e, w1_ref[...], preferred_element_type=jnp.float32) + b1_ref[...]
  h = jnp.maximum(h, 0.0)
  h = jnp.dot(h, w2_ref[...], preferred_element_type=jnp.float32) + b2_ref[...]
  h = jnp.maximum(h, 0.0)
  deep = jnp.dot(h, w3_ref[...], preferred_element_type=jnp.float32)
  first = jnp.sum(linv_ref[...], axis=1, keepdims=True)
  out_ref[...] = first + second + deep + c0_ref[0, 0]


def _tc_fused(e, linv, sel, w1f, b1f, w2f, b2f, w3, c0):
  bm = 2048
  grid = (B // bm,)
  return pl.pallas_call(
      _tc_body,
      grid=grid,
      in_specs=[
          pl.BlockSpec((bm, F * D), lambda i: (i, 0)),
          pl.BlockSpec((bm, F), lambda i: (i, 0)),
          pl.BlockSpec((F * D, D), lambda i: (0, 0)),
          pl.BlockSpec((F * D, H1), lambda i: (0, 0)),
          pl.BlockSpec((1, H1), lambda i: (0, 0)),
          pl.BlockSpec((H1, H2), lambda i: (0, 0)),
          pl.BlockSpec((1, H2), lambda i: (0, 0)),
          pl.BlockSpec((H2, 1), lambda i: (0, 0)),
          pl.BlockSpec((1, 1), lambda i: (0, 0)),
      ],
      out_specs=pl.BlockSpec((bm, 1), lambda i: (i, 0)),
      out_shape=jax.ShapeDtypeStruct((B, 1), jnp.float32),
  )(e, linv, sel, w1f, b1f, w2f, b2f, w3, c0)


def kernel(x, emb_w, lin_w, lin_b, W1, b1, g1, be1, rm1, rv1, W2, b2, g2, be2,
           rm2, rv2, W3, b3):
  offsets = (jnp.arange(F, dtype=x.dtype) * V)[None, :]
  idx2d = (x + offsets).reshape(BF // G, G)

  emb2d = emb_w.reshape(total * D).reshape(total, D)
  e_flat, linv_flat = _sc_gather(idx2d, emb2d, lin_w.reshape(total))
  e = e_flat.reshape(B, F * D)
  linv = linv_flat.reshape(B, F)

  inv1 = g1 / jnp.sqrt(rv1 + EPS)
  w1f = W1 * inv1[None, :]
  b1f = ((b1 - rm1) * inv1 + be1)[None, :]
  inv2 = g2 / jnp.sqrt(rv2 + EPS)
  w2f = W2 * inv2[None, :]
  b2f = ((b2 - rm2) * inv2 + be2)[None, :]
  sel = jnp.tile(jnp.eye(D, dtype=jnp.float32), (F, 1))
  c0 = (lin_b + b3).reshape(1, 1)

  out = _tc_fused(e, linv, sel, w1f, b1f, w2f, b2f, W3, c0)
  return out.reshape(B)
